# baseline (speedup 1.0000x reference)
import functools

import jax
import jax.numpy as jnp
from jax import lax
from jax.experimental import pallas as pl
from jax.experimental.pallas import tpu as pltpu
from jax.experimental.pallas import tpu_sc as plsc

NW = 32
RPW = 8
NSEL = NW * RPW
NBINS = 256
NSLOT = 4
CAP = 128


def _smap_body(feat_ref, knn_ref, out_ref, rmax_ref):
    res = jax.lax.dot_general(
        feat_ref[...], knn_ref[0], (((1,), (1,)), ((), ())),
        preferred_element_type=jnp.float32,
    )
    out_ref[0] = res
    rmax_ref[0] = jnp.max(res, axis=1, keepdims=True)


def _worker_id():
    return lax.axis_index("c") * 16 + lax.axis_index("s")


def _sc_mesh():
    return plsc.VectorSubcoreMesh(core_axis_name="c", subcore_axis_name="s")


def _bin_of(v, lo, sc):
    b = ((v - lo) * sc).astype(jnp.int32)
    return jnp.minimum(jnp.maximum(b, 0), NBINS - 1)


def _make_hist_kernel(K, Q, N):
    nvr = N // 16

    @functools.partial(
        pl.kernel, mesh=_sc_mesh(),
        compiler_params=pltpu.CompilerParams(needs_layout_passes=False),
        out_type=jax.ShapeDtypeStruct((K, NW, NBINS), jnp.int32),
        scratch_types=[
            pltpu.VMEM((K * RPW,), jnp.int32),
            pltpu.VMEM((K * 48,), jnp.float32),
            pltpu.VMEM((RPW, N), jnp.float32),
            pltpu.VMEM((RPW, N), jnp.float32),
            pltpu.VMEM((NSLOT * 16 * NBINS,), jnp.int32),
            pltpu.VMEM((NBINS,), jnp.int32),
            pltpu.SemaphoreType.DMA,
            pltpu.SemaphoreType.DMA,
        ],
    )
    def hist_kernel(smaps2d, ids, params, out, idv, pv, rows0, rows1, h2,
                    merged, sem0, sem1):
        wid = _worker_id()
        lane = lax.iota(jnp.int32, 16)
        ones = jnp.full((16,), 1, jnp.int32)
        zeros = jnp.zeros((16,), jnp.int32)
        rowbufs = (rows0, rows1)
        sems = (sem0, sem1)

        pltpu.sync_copy(ids.at[wid], idv)
        pltpu.sync_copy(params, pv)

        def zero_body(i, _):
            h2[pl.ds(i * 16, 16)] = zeros
            return 0
        lax.fori_loop(0, NSLOT * NBINS, zero_body, 0)

        def gather(k, buf):
            return pltpu.async_copy(
                smaps2d.at[idv.at[pl.ds(k * RPW, RPW)]], rowbufs[buf],
                sems[buf])

        pend = gather(0, 0)
        for k in range(K):
            if k + 1 < K:
                nxt = gather(k + 1, (k + 1) % 2)
            pend.wait()
            rows = rowbufs[k % 2]
            lov_ = pv[pl.ds(k * 48, 16)]
            scv_ = pv[pl.ds(k * 48 + 16, 16)]

            @plsc.parallel_loop(0, RPW * nvr, step=NSLOT, unroll=2)
            def body(t, rows=rows, lov_=lov_, scv_=scv_):
                s = t // nvr
                c = t - s * nvr
                for j in range(NSLOT):
                    v = rows[s, pl.ds((c + j) * 16, 16)]
                    b = _bin_of(v, lov_, scv_)
                    idx = (j * 16 + lane) * NBINS + b
                    plsc.addupdate_scatter(h2, [idx], ones)

            def merge_body(i, _):
                acc = h2[pl.ds(i * 16, 16)]
                h2[pl.ds(i * 16, 16)] = zeros
                for r in range(1, NSLOT * 16):
                    acc = acc + h2[pl.ds(r * NBINS + i * 16, 16)]
                    h2[pl.ds(r * NBINS + i * 16, 16)] = zeros
                merged[pl.ds(i * 16, 16)] = acc
                return 0
            lax.fori_loop(0, NBINS // 16, merge_body, 0)
            pltpu.sync_copy(merged, out.at[k, wid])
            if k + 1 < K:
                pend = nxt

    return hist_kernel


def _make_compact_kernel(K, Q, N):
    nvr = N // 16

    @functools.partial(
        pl.kernel, mesh=_sc_mesh(),
        compiler_params=pltpu.CompilerParams(needs_layout_passes=False),
        out_type=(jax.ShapeDtypeStruct((K, NW, CAP), jnp.float32),
                  jax.ShapeDtypeStruct((K, NW, CAP), jnp.int32)),
        scratch_types=[
            pltpu.VMEM((K * RPW,), jnp.int32),
            pltpu.VMEM((K * RPW * 16,), jnp.int32),
            pltpu.VMEM((K * 16,), jnp.float32),
            pltpu.VMEM((RPW, N), jnp.float32),
            pltpu.VMEM((RPW, N), jnp.float32),
            pltpu.VMEM((CAP,), jnp.float32),
            pltpu.VMEM((CAP,), jnp.int32),
            pltpu.SemaphoreType.DMA,
            pltpu.SemaphoreType.DMA,
        ],
    )
    def compact_kernel(smaps2d, ids, rbase, params, ov, oi, idv, rbv, pv,
                       rows0, rows1, cv, ci, sem0, sem1):
        wid = _worker_id()
        lane = lax.iota(jnp.int32, 16)
        negpad = jnp.full((16,), -3.0, jnp.float32)
        zeros = jnp.zeros((16,), jnp.int32)
        rowbufs = (rows0, rows1)
        sems = (sem0, sem1)

        pltpu.sync_copy(ids.at[wid], idv)
        pltpu.sync_copy(rbase.at[wid], rbv)
        pltpu.sync_copy(params, pv)

        def gather(k, buf):
            return pltpu.async_copy(
                smaps2d.at[idv.at[pl.ds(k * RPW, RPW)]], rowbufs[buf],
                sems[buf])

        pend = gather(0, 0)
        for k in range(K):
            if k + 1 < K:
                nxt = gather(k + 1, (k + 1) % 2)
            pend.wait()
            rows = rowbufs[k % 2]
            lov_ = pv[pl.ds(k * 16, 16)]

            def fill_body(i, _):
                cv[pl.ds(i * 16, 16)] = negpad
                ci[pl.ds(i * 16, 16)] = zeros
                return 0
            lax.fori_loop(0, CAP // 16, fill_body, 0)

            @plsc.parallel_loop(0, RPW * nvr, step=2, unroll=2,
                                carry=jnp.zeros((16,), jnp.int32))
            def body(t, cnt, rows=rows, lov_=lov_, k=k):
                s = t // nvr
                c = t - s * nvr
                rb = rbv[pl.ds((k * RPW + s) * 16, 16)]
                for j in range(2):
                    v = rows[s, pl.ds((c + j) * 16, 16)]
                    m = v >= lov_
                    pos = cnt + plsc.cumsum(m.astype(jnp.int32)) - 1
                    pos = jnp.minimum(pos, CAP - 1)
                    plsc.store_scatter(cv, [pos], v, mask=m)
                    fidx = rb + (c + j) * 16 + lane
                    plsc.store_scatter(ci, [pos], fidx, mask=m)
                    cnt = cnt + plsc.all_reduce_population_count(m)
                return cnt

            pltpu.sync_copy(cv, ov.at[k, wid])
            pltpu.sync_copy(ci, oi.at[k, wid])
            if k + 1 < K:
                pend = nxt

    return compact_kernel


def kernel(feat, knn_feats, bbox, knn_bboxes):
    Q, D = feat.shape
    K, N, _ = knn_feats.shape
    topk = max(int(0.1 * Q), 1)

    feat_norm = feat / jnp.clip(jnp.linalg.norm(feat, axis=1, keepdims=True), 1e-12)
    knn_norm = knn_feats / jnp.clip(jnp.linalg.norm(knn_feats, axis=2, keepdims=True), 1e-12)

    TQ = 256
    smaps, rmax = pl.pallas_call(
        _smap_body,
        grid=(K, Q // TQ),
        in_specs=[
            pl.BlockSpec((TQ, D), lambda k, q: (q, 0)),
            pl.BlockSpec((1, N, D), lambda k, q: (k, 0, 0)),
        ],
        out_specs=[
            pl.BlockSpec((1, TQ, N), lambda k, q: (k, q, 0)),
            pl.BlockSpec((1, TQ, 1), lambda k, q: (k, q, 0)),
        ],
        out_shape=[
            jax.ShapeDtypeStruct((K, Q, N), jnp.float32),
            jax.ShapeDtypeStruct((K, Q, 1), jnp.float32),
        ],
    )(feat_norm, knn_norm)
    rmax = rmax.reshape(K, Q)

    selmax, selrow = jax.lax.top_k(rmax, NSEL)
    mstar = selmax[:, NSEL - 1]

    lsel = jnp.sort(selrow, axis=1).astype(jnp.int32)
    gsel = lsel + (jnp.arange(K, dtype=jnp.int32) * Q)[:, None]
    ids = gsel.reshape(K, NW, RPW).transpose(1, 0, 2).reshape(NW, K * RPW)
    rbase = jnp.broadcast_to(
        (lsel * N).reshape(K, NW, RPW).transpose(1, 0, 2)[..., None],
        (NW, K, RPW, 16)).reshape(NW, K * RPW * 16)

    smaps2d = smaps.reshape(K * Q, N)
    params = jnp.broadcast_to(mstar[:, None], (K, 16)).reshape(K * 16)

    cand_v, cand_i = _make_compact_kernel(K, Q, N)(smaps2d, ids, rbase, params)

    top_vals, pos = jax.lax.top_k(cand_v.reshape(K, NW * CAP), topk)
    top_inds = jnp.take_along_axis(cand_i.reshape(K, NW * CAP), pos, axis=1)
    qi = top_inds // N
    ki = top_inds % N
    qbox = jnp.take(bbox, qi, axis=0)
    kbox = jnp.take_along_axis(knn_bboxes, ki[..., None], axis=1)
    pairs = jnp.concatenate([qbox, kbox], axis=-1)
    return (bbox, pairs, top_vals)
